# baseline (speedup 1.0000x reference)
import functools
import jax
import jax.numpy as jnp
from jax import lax
from jax.experimental import pallas as pl
from jax.experimental.pallas import tpu as pltpu
from jax.experimental.pallas import tpu_sc as plsc

NN = 10000
DD = 64
EE = 320000
NT = 16
SUB = 128
EPT = EE // NT
NBUF = 4
NCH = 160
EPT_PAD = NCH * SUB
E_PAD = EPT_PAD * NT
EXT = 512
WRB = 128
NSL = 640
EPS = 0.1


def _rsqrt_nr(x):
    i = lax.bitcast_convert_type(x, jnp.int32)
    i = 0x5F3759DF - lax.shift_right_arithmetic(i, 1)
    y = lax.bitcast_convert_type(i, jnp.float32)
    for _ in range(3):
        y = y * (1.5 - 0.5 * x * y * y)
    return y


def _sc_layer_body(hcur, hraw, a_hbm, b_hbm, src_hbm, dst_hbm, out,
                   nrm_v, sidx_v, didx_v, av_v, bv_v, coefc_v, rows_v,
                   nsl_v, h_sh, acc_sh, nrm_sh, a_sh, b_sh,
                   sem_i, sem_a, sem_b, sem_g, sem_s, sem_p):
    c = lax.axis_index("c")
    t = lax.axis_index("s")
    cN = c * NN
    half_sign = 1.0 - 2.0 * c.astype(jnp.float32)
    ebase = t * EPT_PAD

    def _zrow(e, _):
        for g in range(4):
            rows_v[0, e, pl.ds(g * 16, 16)] = jnp.zeros((16,), jnp.float32)
        return 0
    lax.fori_loop(0, SUB, _zrow, 0)

    def _znsl(i, _):
        nsl_v[pl.ds(i * 16, 16)] = jnp.zeros((16,), jnp.float32)
        return 0
    lax.fori_loop(0, NSL // 16, _znsl, 0)

    def _stage(base, sz):
        sub = pl.ds(base, sz)
        d0 = pltpu.async_copy(hcur.at[pl.ds(cN + base, sz)], h_sh.at[sub],
                              sem_g.at[0])
        d1 = pltpu.async_copy(a_hbm.at[sub], a_sh.at[sub], sem_a.at[0])
        d2 = pltpu.async_copy(b_hbm.at[sub], b_sh.at[sub], sem_b.at[0])
        d3 = pltpu.async_copy(nsl_v.at[pl.ds(0, sz)], nrm_sh.at[sub],
                              sem_p.at[0])
        for d in (d0, d1, d2, d3):
            d.wait()

    def _acc_zero_chunk(r0, sz):
        pltpu.sync_copy(rows_v.at[0, pl.ds(0, sz)], acc_sh.at[pl.ds(r0, sz)])

    @pl.when(t < 15)
    def _():
        _stage(t * NSL, NSL)

        def _k(k, _):
            _acc_zero_chunk(t * NSL + k * WRB, WRB)
            return 0
        lax.fori_loop(0, 5, _k, 0)

    @pl.when(t == 15)
    def _():
        _stage(9600, 400)

        def _k(k, _):
            _acc_zero_chunk(9600 + k * WRB, WRB)
            return 0
        lax.fori_loop(0, 3, _k, 0)
        _acc_zero_chunk(9984, 16)

    plsc.subcore_barrier()

    def _issue_idx(jc, p):
        off = ebase + jc * SUB
        di = pltpu.async_copy(src_hbm.at[pl.ds(off, SUB)], sidx_v.at[p],
                              sem_i.at[p])
        dj = pltpu.async_copy(dst_hbm.at[pl.ds(off, SUB)], didx_v.at[p],
                              sem_i.at[p])
        return di, dj

    def _wait_idx_ring(p):
        pltpu.make_async_copy(src_hbm.at[pl.ds(0, SUB)], sidx_v.at[p],
                              sem_i.at[p]).wait()
        pltpu.make_async_copy(dst_hbm.at[pl.ds(0, SUB)], didx_v.at[p],
                              sem_i.at[p]).wait()

    def _issue_ab(p):
        da = pltpu.async_copy(a_sh.at[didx_v.at[p]], av_v.at[p],
                              sem_a.at[p])
        db = pltpu.async_copy(b_sh.at[sidx_v.at[p]], bv_v.at[p],
                              sem_b.at[p])
        return da, db

    def _wait_ab_ring(p):
        pltpu.make_async_copy(a_hbm.at[pl.ds(0, SUB)], av_v.at[p],
                              sem_a.at[p]).wait()
        pltpu.make_async_copy(b_hbm.at[pl.ds(0, SUB)], bv_v.at[p],
                              sem_b.at[p]).wait()

    def _issue_rows(p):
        return pltpu.async_copy(h_sh.at[sidx_v.at[p]], rows_v.at[p],
                                sem_g.at[p])

    def _wait_rows_ring(p):
        pltpu.make_async_copy(hcur.at[pl.ds(0, SUB)], rows_v.at[p],
                              sem_g.at[p]).wait()

    def _issue_scat_rows(p):
        return pltpu.async_copy(rows_v.at[p], acc_sh.at[didx_v.at[p]],
                                sem_s.at[p], add=True)

    def _wait_scat_rows_ring(p):
        pltpu.make_async_copy(rows_v.at[p], acc_sh.at[pl.ds(0, SUB)],
                              sem_s.at[p]).wait()

    def _issue_scat_sc(p):
        return pltpu.async_copy(coefc_v.at[p, pl.ds(0, SUB)],
                                nrm_sh.at[didx_v.at[p]],
                                sem_p.at[p], add=True)

    def _wait_scat_sc_ring(p):
        pltpu.make_async_copy(coefc_v.at[p, pl.ds(0, SUB)],
                              nrm_sh.at[pl.ds(0, SUB)],
                              sem_p.at[p]).wait()

    def _score(jc, p, g):
        sl = pl.ds(g * 16, 16)
        x = jnp.clip(av_v[p, sl] + bv_v[p, sl], -20.0, 20.0)
        ex = jnp.exp(2.0 * x)
        sub = (ex - 1.0) / (ex + 1.0)
        s = 0.5 + 0.5 * half_sign * sub
        pos = jc * SUB + g * 16 + lax.iota(jnp.int32, 16)
        return jnp.where(pos < EPT, s, 0.0)

    def _p1_proc(jc, p):
        def _grp(g2, _):
            for u in range(2):
                g = g2 * 2 + u
                coefc_v[p, pl.ds(g * 16, 16)] = _score(jc, p, g)
            return 0
        lax.fori_loop(0, SUB // 32, _grp, 0)

    def _p1_slot(jc, u, wait_sc, wait_ab_real, wait_idx_real, descs):
        p = u % NBUF
        if wait_sc:
            _wait_scat_sc_ring((u - 2) % NBUF)
        descs[("i", (u + 2) % NBUF)] = _issue_idx(jc + 2, (u + 2) % NBUF)
        if wait_ab_real:
            for d in descs[("ab", p)]:
                d.wait()
        else:
            _wait_ab_ring(p)
        _p1_proc(jc, p)
        _issue_scat_sc(p)
        if wait_idx_real:
            for d in descs[("i", (u + 1) % NBUF)]:
                d.wait()
        else:
            _wait_idx_ring((u + 1) % NBUF)
        descs[("ab", (u + 1) % NBUF)] = _issue_ab((u + 1) % NBUF)

    with jax.named_scope("p1_scores"):
        descs = {}
        descs[("i", 0)] = _issue_idx(0, 0)
        descs[("i", 1)] = _issue_idx(1, 1)
        for d in descs[("i", 0)]:
            d.wait()
        descs[("ab", 0)] = _issue_ab(0)
        for u in range(NBUF):
            _p1_slot(u, u, u >= 2, True, True, descs)

        def _p1_it(it, _):
            jc0 = it * NBUF
            for u in range(NBUF):
                _p1_slot(jc0 + u, u, True, False, False, {})
            return 0
        lax.fori_loop(1, NCH // NBUF, _p1_it, 0)
        _wait_idx_ring(1)
        _wait_ab_ring(0)
        _wait_scat_sc_ring(2)
        _wait_scat_sc_ring(3)

    plsc.subcore_barrier()

    def _finalize(base, sz):
        dstsl = nsl_v.at[pl.ds(0, sz)] if sz < NSL else nsl_v
        pltpu.sync_copy(nrm_sh.at[pl.ds(base, sz)], dstsl)

        def _nr(i, _):
            x = jnp.maximum(nsl_v[pl.ds(i * 16, 16)], 1.0)
            nsl_v[pl.ds(i * 16, 16)] = _rsqrt_nr(x)
            return 0
        lax.fori_loop(0, sz // 16, _nr, 0)
        pltpu.sync_copy(dstsl, nrm_sh.at[pl.ds(base, sz)])

    @pl.when(t < 15)
    def _():
        _finalize(t * NSL, NSL)

    @pl.when(t == 15)
    def _():
        _finalize(9600, 400)

    plsc.subcore_barrier()
    pltpu.sync_copy(nrm_sh, nrm_v.at[pl.ds(0, NN)])

    def _p2_proc(jc, p):
        def _g(g2, _):
            for u in range(2):
                g = g2 * 2 + u
                s = _score(jc, p, g)
                sv = sidx_v[p, pl.ds(g * 16, 16)]
                ns = plsc.load_gather(nrm_v, [sv])
                coefc_v[p, pl.ds(g * 16, 16)] = s * ns
            return 0
        lax.fori_loop(0, SUB // 32, _g, 0)
        _wait_rows_ring(p)

        def _s4(e4, _):
            for u in range(4):
                e = e4 * 4 + u
                cf = coefc_v[p, pl.ds(e, 16)][0]
                for gg in range(4):
                    sl = pl.ds(gg * 16, 16)
                    rows_v[p, e, sl] = rows_v[p, e, sl] * cf
            return 0
        lax.fori_loop(0, SUB // 4, _s4, 0)

    def _p2_slot(jc, u, wait_sc, wait_ab_real, wait_idx_real, descs):
        p = u % NBUF
        if wait_sc:
            _wait_scat_rows_ring((u - 2) % NBUF)
        descs[("i", (u + 2) % NBUF)] = _issue_idx(jc + 2, (u + 2) % NBUF)
        if wait_ab_real:
            for d in descs[("ab", p)]:
                d.wait()
        else:
            _wait_ab_ring(p)
        _p2_proc(jc, p)
        _issue_scat_rows(p)
        if wait_idx_real:
            for d in descs[("i", (u + 1) % NBUF)]:
                d.wait()
        else:
            _wait_idx_ring((u + 1) % NBUF)
        descs[("ab", (u + 1) % NBUF)] = _issue_ab((u + 1) % NBUF)
        _issue_rows((u + 1) % NBUF)

    with jax.named_scope("p2_propagate"):
        descs = {}
        descs[("i", 0)] = _issue_idx(0, 0)
        descs[("i", 1)] = _issue_idx(1, 1)
        for d in descs[("i", 0)]:
            d.wait()
        descs[("ab", 0)] = _issue_ab(0)
        _issue_rows(0)
        for u in range(NBUF):
            _p2_slot(u, u, u >= 2, True, True, descs)

        def _p2_it(it, _):
            jc0 = it * NBUF
            for u in range(NBUF):
                _p2_slot(jc0 + u, u, True, False, False, {})
            return 0
        lax.fori_loop(1, NCH // NBUF, _p2_it, 0)
        _wait_idx_ring(1)
        _wait_ab_ring(0)
        _wait_rows_ring(0)
        _wait_scat_rows_ring(2)
        _wait_scat_rows_ring(3)

    plsc.subcore_barrier()

    def _wb_chunk(r0, sz):
        accsl = rows_v.at[0, pl.ds(0, sz)]
        rawsl = rows_v.at[1, pl.ds(0, sz)]
        d0 = pltpu.async_copy(acc_sh.at[pl.ds(r0, sz)], accsl, sem_g.at[0])
        d1 = pltpu.async_copy(hraw.at[pl.ds(cN + r0, sz)], rawsl,
                              sem_g.at[1])
        d0.wait()
        d1.wait()

        def _blend(e, _):
            nr = (1.0 - EPS) * nrm_v[pl.ds(r0 + e, 16)][0]
            for g in range(4):
                sl = pl.ds(g * 16, 16)
                rows_v[0, e, sl] = (nr * rows_v[0, e, sl]
                                    + EPS * rows_v[1, e, sl])
            return 0
        lax.fori_loop(0, sz, _blend, 0)
        pltpu.sync_copy(accsl, out.at[pl.ds(cN + r0, sz)])

    @pl.when(t < 15)
    def _():
        def _k(k, _):
            _wb_chunk(t * NSL + k * WRB, WRB)
            return 0
        lax.fori_loop(0, 5, _k, 0)

    @pl.when(t == 15)
    def _():
        def _k(k, _):
            _wb_chunk(9600 + k * WRB, WRB)
            return 0
        lax.fori_loop(0, 3, _k, 0)
        _wb_chunk(9984, 16)


@jax.jit
def _sc_layer(hcur, hraw, a, b, src_pad, dst_pad):
    mesh = plsc.VectorSubcoreMesh(core_axis_name="c", subcore_axis_name="s")
    f32 = jnp.float32
    return pl.kernel(
        _sc_layer_body,
        out_type=jax.ShapeDtypeStruct((2 * NN, DD), f32),
        mesh=mesh,
        compiler_params=pltpu.CompilerParams(needs_layout_passes=False,
                                             use_tc_tiling_on_sc=False),
        scratch_types=[
            pltpu.VMEM((NN + 16,), f32),
            pltpu.VMEM((NBUF, SUB), jnp.int32),
            pltpu.VMEM((NBUF, SUB), jnp.int32),
            pltpu.VMEM((NBUF, SUB), f32),
            pltpu.VMEM((NBUF, SUB), f32),
            pltpu.VMEM((NBUF, SUB + 16), f32),
            pltpu.VMEM((NBUF, SUB, DD), f32),
            pltpu.VMEM((NSL,), f32),
            pltpu.VMEM_SHARED((NN, DD), f32),
            pltpu.VMEM_SHARED((NN, DD), f32),
            pltpu.VMEM_SHARED((NN,), f32),
            pltpu.VMEM_SHARED((NN,), f32),
            pltpu.VMEM_SHARED((NN,), f32),
            pltpu.SemaphoreType.DMA((NBUF,)),
            pltpu.SemaphoreType.DMA((NBUF,)),
            pltpu.SemaphoreType.DMA((NBUF,)),
            pltpu.SemaphoreType.DMA((NBUF,)),
            pltpu.SemaphoreType.DMA((NBUF,)),
            pltpu.SemaphoreType.DMA((NBUF,)),
        ],
    )(hcur, hraw, a, b, src_pad, dst_pad)


def _front_body(h_ref, wre_ref, bre_ref, wir_ref, bir_ref, w2_ref, bg_ref,
                hout_ref, ab_ref):
    h = h_ref[...]
    re = jnp.maximum(jnp.dot(h, wre_ref[...],
                             preferred_element_type=jnp.float32)
                     + bre_ref[...], 0.0)
    ir = jnp.maximum(jnp.dot(h, wir_ref[...],
                             preferred_element_type=jnp.float32)
                     + bir_ref[...], 0.0)
    hout_ref[0:NN, :] = re
    hout_ref[NN:2 * NN, :] = ir
    hcat = jnp.concatenate([re, ir], axis=1)
    ab_ref[...] = jnp.dot(hcat, w2_ref[...],
                          preferred_element_type=jnp.float32) + bg_ref[...]


def _gate_body(hio_ref, w2_ref, bg_ref, ab_ref):
    hcat = jnp.concatenate([hio_ref[0:NN, :], hio_ref[NN:2 * NN, :]], axis=1)
    ab_ref[...] = jnp.dot(hcat, w2_ref[...],
                          preferred_element_type=jnp.float32) + bg_ref[...]


def _back_body(hio_ref, wc_ref, bc_ref, rl_ref, il_ref):
    rl_ref[...] = jnp.dot(hio_ref[0:NN, :], wc_ref[...],
                          preferred_element_type=jnp.float32) + bc_ref[...]
    il_ref[...] = jnp.dot(hio_ref[NN:2 * NN, :], wc_ref[...],
                          preferred_element_type=jnp.float32) + bc_ref[...]


def _gate_weights(Wg, bg):
    w2 = jnp.concatenate([Wg[:2 * DD], Wg[2 * DD:]], axis=1)
    w2 = jnp.pad(w2, ((0, 0), (0, 6)))
    bg8 = jnp.zeros((1, 8), jnp.float32).at[0, 0].set(bg[0])
    return w2, bg8


def kernel(h, edge_index, Wre, bre, Wir, bir, Wg0, bg0, Wg1, bg1, Wc, bc):
    f32 = jnp.float32

    def _tile_pad(x):
        y = jnp.pad(x.reshape(NT, EPT),
                    ((0, 0), (0, EPT_PAD - EPT))).reshape(-1)
        return jnp.pad(y, (0, EXT))

    src_pad = _tile_pad(edge_index[0])
    dst_pad = _tile_pad(edge_index[1])

    w2g0, bg0v = _gate_weights(Wg0, bg0)
    w2g1, bg1v = _gate_weights(Wg1, bg1)

    hraw, ab0 = pl.pallas_call(
        _front_body,
        out_shape=[
            jax.ShapeDtypeStruct((2 * NN, DD), f32),
            jax.ShapeDtypeStruct((NN, 8), f32),
        ],
    )(h, Wre, bre.reshape(1, DD), Wir, bir.reshape(1, DD), w2g0, bg0v)

    out1 = _sc_layer(hraw, hraw, ab0[:, 0], ab0[:, 1], src_pad, dst_pad)

    ab1 = pl.pallas_call(
        _gate_body,
        out_shape=jax.ShapeDtypeStruct((NN, 8), f32),
    )(out1, w2g1, bg1v)

    out2 = _sc_layer(out1, hraw, ab1[:, 0], ab1[:, 1], src_pad, dst_pad)

    re2 = out2[:NN]
    ir2 = out2[NN:]
    re_logits, ir_logits = pl.pallas_call(
        _back_body,
        out_shape=[
            jax.ShapeDtypeStruct((NN, DD), f32),
            jax.ShapeDtypeStruct((NN, DD), f32),
        ],
    )(out2, Wc, bc.reshape(1, DD))
    return (re_logits, ir_logits, re2, ir2)


# speedup vs baseline: 1.1787x; 1.1787x over previous
import functools
import jax
import jax.numpy as jnp
from jax import lax
from jax.experimental import pallas as pl
from jax.experimental.pallas import tpu as pltpu
from jax.experimental.pallas import tpu_sc as plsc

NN = 10000
DD = 64
EE = 320000
NT = 16
SUB = 128
KSUB = 20
SUP = SUB * KSUB
NSUP = 8
EPT = EE // NT
EPT_PAD = SUP * NSUP
E_PAD = EPT_PAD * NT
WRB = 128
NBUF = 3
NSL = 640
EPS = 0.1


def _rsqrt_nr(x):
    i = lax.bitcast_convert_type(x, jnp.int32)
    i = 0x5F3759DF - lax.shift_right_arithmetic(i, 1)
    y = lax.bitcast_convert_type(i, jnp.float32)
    for _ in range(3):
        y = y * (1.5 - 0.5 * x * y * y)
    return y


def _sc_layer_body(hcur, hraw, a_hbm, b_hbm, src_hbm, dst_hbm, out,
                   nrm_v, s2_v, d2_v, didx_v, av_v, bv_v, coefc_v, rows_v,
                   nsl_v, h_sh, acc_sh, nrm_sh, a_sh, b_sh,
                   sem_a, sem_b, sem_g, sem_s, sem_p):
    c = lax.axis_index("c")
    t = lax.axis_index("s")
    cN = c * NN
    half_sign = 1.0 - 2.0 * c.astype(jnp.float32)
    ebase = t * EPT_PAD

    def _zrow(e, _):
        for g in range(4):
            rows_v[0, e, pl.ds(g * 16, 16)] = jnp.zeros((16,), jnp.float32)
        return 0
    lax.fori_loop(0, SUB, _zrow, 0)

    def _znsl(i, _):
        nsl_v[pl.ds(i * 16, 16)] = jnp.zeros((16,), jnp.float32)
        return 0
    lax.fori_loop(0, NSL // 16, _znsl, 0)

    def _stage(base, sz):
        sub = pl.ds(base, sz)
        d0 = pltpu.async_copy(hcur.at[pl.ds(cN + base, sz)], h_sh.at[sub],
                              sem_g.at[0])
        d1 = pltpu.async_copy(a_hbm.at[sub], a_sh.at[sub], sem_a.at[0])
        d2 = pltpu.async_copy(b_hbm.at[sub], b_sh.at[sub], sem_b.at[0])
        d3 = pltpu.async_copy(nsl_v.at[pl.ds(0, sz)], nrm_sh.at[sub],
                              sem_p.at[0])
        for d in (d0, d1, d2, d3):
            d.wait()

    def _acc_zero_chunk(r0, sz):
        pltpu.sync_copy(rows_v.at[0, pl.ds(0, sz)], acc_sh.at[pl.ds(r0, sz)])

    @pl.when(t < 15)
    def _():
        _stage(t * NSL, NSL)

        def _k(k, _):
            _acc_zero_chunk(t * NSL + k * WRB, WRB)
            return 0
        lax.fori_loop(0, 5, _k, 0)

    @pl.when(t == 15)
    def _():
        _stage(9600, 400)

        def _k(k, _):
            _acc_zero_chunk(9600 + k * WRB, WRB)
            return 0
        lax.fori_loop(0, 3, _k, 0)
        _acc_zero_chunk(9984, 16)

    plsc.subcore_barrier()

    def _load_super(si):
        off = ebase + si * SUP
        pltpu.sync_copy(src_hbm.at[pl.ds(off, SUP)], s2_v)
        pltpu.sync_copy(dst_hbm.at[pl.ds(off, SUP)], d2_v)

    def _idxcopy(j, p):
        def _cp(g, _):
            didx_v[p, pl.ds(g * 16, 16)] = d2_v[pl.ds(j * SUB + g * 16, 16)]
            return 0
        lax.fori_loop(0, SUB // 16, _cp, 0)

    def _ab_streams(j, p):
        isl = pl.ds(j * SUB, SUB)
        da = pltpu.async_copy(a_sh.at[d2_v.at[isl]], av_v.at[p],
                              sem_a.at[p])
        db = pltpu.async_copy(b_sh.at[s2_v.at[isl]], bv_v.at[p],
                              sem_b.at[p])
        return da, db

    def _score(si, j, p, g):
        sl = pl.ds(g * 16, 16)
        x = jnp.clip(av_v[p, sl] + bv_v[p, sl], -20.0, 20.0)
        ex = jnp.exp(2.0 * x)
        sub = (ex - 1.0) / (ex + 1.0)
        s = 0.5 + 0.5 * half_sign * sub
        pos = si * SUP + j * SUB + g * 16 + lax.iota(jnp.int32, 16)
        return jnp.where(pos < EPT, s, 0.0)

    def _p1_super(si, _):
        _load_super(si)
        ab = {}
        sc = {}
        for j in range(KSUB):
            p = j % NBUF
            if j >= NBUF:
                sc[j - NBUF].wait()
            _idxcopy(j, p)
            ab[j] = _ab_streams(j, p)

            def _proc(jq, q):
                for d in ab[jq]:
                    d.wait()

                def _grp(g2, _, jq=jq, q=q):
                    for u in range(2):
                        g = g2 * 2 + u
                        s = _score(si, jq, q, g)
                        coefc_v[q, pl.ds(g * 16, 16)] = s
                    return 0
                lax.fori_loop(0, SUB // 32, _grp, 0)
                sc[jq] = pltpu.async_copy(coefc_v.at[q, pl.ds(0, SUB)],
                                          nrm_sh.at[didx_v.at[q]],
                                          sem_p.at[q], add=True)

            if j >= 1:
                _proc(j - 1, (j - 1) % NBUF)
        _proc(KSUB - 1, (KSUB - 1) % NBUF)
        for j in range(KSUB - NBUF, KSUB):
            sc[j].wait()
        return 0

    with jax.named_scope("p1_scores"):
        lax.fori_loop(0, NSUP, _p1_super, 0)

    plsc.subcore_barrier()

    def _finalize(base, sz):
        dstsl = nsl_v.at[pl.ds(0, sz)] if sz < NSL else nsl_v
        pltpu.sync_copy(nrm_sh.at[pl.ds(base, sz)], dstsl)

        def _nr(i, _):
            x = jnp.maximum(nsl_v[pl.ds(i * 16, 16)], 1.0)
            nsl_v[pl.ds(i * 16, 16)] = _rsqrt_nr(x)
            return 0
        lax.fori_loop(0, sz // 16, _nr, 0)
        pltpu.sync_copy(dstsl, nrm_sh.at[pl.ds(base, sz)])

    @pl.when(t < 15)
    def _():
        _finalize(t * NSL, NSL)

    @pl.when(t == 15)
    def _():
        _finalize(9600, 400)

    plsc.subcore_barrier()
    pltpu.sync_copy(nrm_sh, nrm_v.at[pl.ds(0, NN)])

    def _p2_super(si, _):
        _load_super(si)
        ab = {}
        gd = {}
        sd = {}

        def _proc(jq, q):
            for d in ab[jq]:
                d.wait()

            def _g(g2, _, jq=jq, q=q):
                for u in range(2):
                    g = g2 * 2 + u
                    s = _score(si, jq, q, g)
                    sv = s2_v[pl.ds(jq * SUB + g * 16, 16)]
                    ns = plsc.load_gather(nrm_v, [sv])
                    coefc_v[q, pl.ds(g * 16, 16)] = s * ns
                return 0
            lax.fori_loop(0, SUB // 32, _g, 0)
            gd[jq].wait()

            def _s4(e4, _, q=q):
                for u in range(4):
                    e = e4 * 4 + u
                    cf = coefc_v[q, pl.ds(e, 16)][0]
                    for gg in range(4):
                        sl = pl.ds(gg * 16, 16)
                        rows_v[q, e, sl] = rows_v[q, e, sl] * cf
                return 0
            lax.fori_loop(0, SUB // 4, _s4, 0)
            sd[jq] = pltpu.async_copy(rows_v.at[q], acc_sh.at[didx_v.at[q]],
                                      sem_s.at[q], add=True)

        for j in range(KSUB):
            p = j % NBUF
            if j >= NBUF:
                sd[j - NBUF].wait()
            _idxcopy(j, p)
            ab[j] = _ab_streams(j, p)
            gd[j] = pltpu.async_copy(h_sh.at[s2_v.at[pl.ds(j * SUB, SUB)]],
                                     rows_v.at[p], sem_g.at[p])
            if j >= 1:
                _proc(j - 1, (j - 1) % NBUF)
        _proc(KSUB - 1, (KSUB - 1) % NBUF)
        for j in range(KSUB - NBUF, KSUB):
            sd[j].wait()
        return 0

    with jax.named_scope("p2_propagate"):
        lax.fori_loop(0, NSUP, _p2_super, 0)

    plsc.subcore_barrier()

    def _wb_chunk(r0, sz):
        accsl = rows_v.at[0, pl.ds(0, sz)]
        rawsl = rows_v.at[1, pl.ds(0, sz)]
        d0 = pltpu.async_copy(acc_sh.at[pl.ds(r0, sz)], accsl, sem_g.at[0])
        d1 = pltpu.async_copy(hraw.at[pl.ds(cN + r0, sz)], rawsl,
                              sem_g.at[1])
        d0.wait()
        d1.wait()

        def _blend(e, _):
            nr = (1.0 - EPS) * nrm_v[pl.ds(r0 + e, 16)][0]
            for g in range(4):
                sl = pl.ds(g * 16, 16)
                rows_v[0, e, sl] = (nr * rows_v[0, e, sl]
                                    + EPS * rows_v[1, e, sl])
            return 0
        lax.fori_loop(0, sz, _blend, 0)
        pltpu.sync_copy(accsl, out.at[pl.ds(cN + r0, sz)])

    @pl.when(t < 15)
    def _():
        def _k(k, _):
            _wb_chunk(t * NSL + k * WRB, WRB)
            return 0
        lax.fori_loop(0, 5, _k, 0)

    @pl.when(t == 15)
    def _():
        def _k(k, _):
            _wb_chunk(9600 + k * WRB, WRB)
            return 0
        lax.fori_loop(0, 3, _k, 0)
        _wb_chunk(9984, 16)


@jax.jit
def _sc_layer(hcur, hraw, a, b, src_pad, dst_pad):
    mesh = plsc.VectorSubcoreMesh(core_axis_name="c", subcore_axis_name="s")
    f32 = jnp.float32
    return pl.kernel(
        _sc_layer_body,
        out_type=jax.ShapeDtypeStruct((2 * NN, DD), f32),
        mesh=mesh,
        compiler_params=pltpu.CompilerParams(needs_layout_passes=False,
                                             use_tc_tiling_on_sc=False),
        scratch_types=[
            pltpu.VMEM((NN + 16,), f32),
            pltpu.VMEM((SUP,), jnp.int32),
            pltpu.VMEM((SUP,), jnp.int32),
            pltpu.VMEM((NBUF, SUB), jnp.int32),
            pltpu.VMEM((NBUF, SUB), f32),
            pltpu.VMEM((NBUF, SUB), f32),
            pltpu.VMEM((NBUF, SUB + 16), f32),
            pltpu.VMEM((NBUF, SUB, DD), f32),
            pltpu.VMEM((NSL,), f32),
            pltpu.VMEM_SHARED((NN, DD), f32),
            pltpu.VMEM_SHARED((NN, DD), f32),
            pltpu.VMEM_SHARED((NN,), f32),
            pltpu.VMEM_SHARED((NN,), f32),
            pltpu.VMEM_SHARED((NN,), f32),
            pltpu.SemaphoreType.DMA((NBUF,)),
            pltpu.SemaphoreType.DMA((NBUF,)),
            pltpu.SemaphoreType.DMA((NBUF,)),
            pltpu.SemaphoreType.DMA((NBUF,)),
            pltpu.SemaphoreType.DMA((NBUF,)),
        ],
    )(hcur, hraw, a, b, src_pad, dst_pad)


def _front_body(h_ref, wre_ref, bre_ref, wir_ref, bir_ref, w2_ref, bg_ref,
                hout_ref, ab_ref):
    h = h_ref[...]
    re = jnp.maximum(jnp.dot(h, wre_ref[...],
                             preferred_element_type=jnp.float32)
                     + bre_ref[...], 0.0)
    ir = jnp.maximum(jnp.dot(h, wir_ref[...],
                             preferred_element_type=jnp.float32)
                     + bir_ref[...], 0.0)
    hout_ref[0:NN, :] = re
    hout_ref[NN:2 * NN, :] = ir
    hcat = jnp.concatenate([re, ir], axis=1)
    ab_ref[...] = jnp.dot(hcat, w2_ref[...],
                          preferred_element_type=jnp.float32) + bg_ref[...]


def _gate_body(hio_ref, w2_ref, bg_ref, ab_ref):
    hcat = jnp.concatenate([hio_ref[0:NN, :], hio_ref[NN:2 * NN, :]], axis=1)
    ab_ref[...] = jnp.dot(hcat, w2_ref[...],
                          preferred_element_type=jnp.float32) + bg_ref[...]


def _back_body(hio_ref, wc_ref, bc_ref, rl_ref, il_ref):
    rl_ref[...] = jnp.dot(hio_ref[0:NN, :], wc_ref[...],
                          preferred_element_type=jnp.float32) + bc_ref[...]
    il_ref[...] = jnp.dot(hio_ref[NN:2 * NN, :], wc_ref[...],
                          preferred_element_type=jnp.float32) + bc_ref[...]


def _gate_weights(Wg, bg):
    w2 = jnp.concatenate([Wg[:2 * DD], Wg[2 * DD:]], axis=1)
    w2 = jnp.pad(w2, ((0, 0), (0, 6)))
    bg8 = jnp.zeros((1, 8), jnp.float32).at[0, 0].set(bg[0])
    return w2, bg8


def kernel(h, edge_index, Wre, bre, Wir, bir, Wg0, bg0, Wg1, bg1, Wc, bc):
    f32 = jnp.float32

    def _tile_pad(x):
        return jnp.pad(x.reshape(NT, EPT),
                       ((0, 0), (0, EPT_PAD - EPT))).reshape(-1)

    src_pad = _tile_pad(edge_index[0])
    dst_pad = _tile_pad(edge_index[1])

    w2g0, bg0v = _gate_weights(Wg0, bg0)
    w2g1, bg1v = _gate_weights(Wg1, bg1)

    hraw, ab0 = pl.pallas_call(
        _front_body,
        out_shape=[
            jax.ShapeDtypeStruct((2 * NN, DD), f32),
            jax.ShapeDtypeStruct((NN, 8), f32),
        ],
    )(h, Wre, bre.reshape(1, DD), Wir, bir.reshape(1, DD), w2g0, bg0v)

    out1 = _sc_layer(hraw, hraw, ab0[:, 0], ab0[:, 1], src_pad, dst_pad)

    ab1 = pl.pallas_call(
        _gate_body,
        out_shape=jax.ShapeDtypeStruct((NN, 8), f32),
    )(out1, w2g1, bg1v)

    out2 = _sc_layer(out1, hraw, ab1[:, 0], ab1[:, 1], src_pad, dst_pad)

    re2 = out2[:NN]
    ir2 = out2[NN:]
    re_logits, ir_logits = pl.pallas_call(
        _back_body,
        out_shape=[
            jax.ShapeDtypeStruct((NN, DD), f32),
            jax.ShapeDtypeStruct((NN, DD), f32),
        ],
    )(out2, Wc, bc.reshape(1, DD))
    return (re_logits, ir_logits, re2, ir2)
